# baseline (speedup 1.0000x reference)
import functools

import jax
import jax.numpy as jnp
from jax import lax
from jax.experimental import pallas as pl
from jax.experimental.pallas import tpu as pltpu
from jax.experimental.pallas import tpu_sc as plsc

_NC = 2
_NS = 16
_NW = _NC * _NS
_L = 16

_B = 4
_N = 4096
_M = 4096

_TC_PTS = 3584
_SC_PTS = _M - _TC_PTS

_TM = 896
_PT = _TC_PTS // _TM
_VCH = 1024

_CHUNKS = _NW // _B
_NPTS = _SC_PTS // _CHUNKS if _SC_PTS else 0
_VBLK = 8
_NBLK = _N // (_L * _VBLK)


def _round_bf16(v):
    bits = lax.bitcast_convert_type(v, jnp.int32)
    lsb = lax.shift_right_logical(bits, 16) & 1
    rounded = (bits + 0x7FFF + lsb) & jnp.int32(-65536)
    return lax.bitcast_convert_type(rounded, jnp.float32)


def _tree_min(vals):
    while len(vals) > 1:
        vals = [jnp.minimum(vals[2 * i], vals[2 * i + 1])
                for i in range(len(vals) // 2)] + vals[len(vals) & ~1:]
    return vals[0]


def _tc_body(vf_ref, pf_ref, out_ref):
    vff = vf_ref[0]
    pft = pf_ref[0]
    vrr = _round_bf16(vff)
    prr = _round_bf16(pft)

    v2 = jnp.sum(vff * vff, axis=0, keepdims=True)

    v2hi = _round_bf16(v2)
    v2lo = v2 - v2hi
    vb = jnp.concatenate([vrr, v2hi, v2lo],
                         axis=0).astype(jnp.bfloat16)
    pb_rows = jnp.concatenate([-2.0 * prr,
                               jnp.ones((2, _TM), jnp.float32)], axis=0)
    pb = jnp.transpose(pb_rows).astype(jnp.bfloat16)
    mns = []
    for c in range(_N // _VCH):
        gc = lax.dot_general(pb, vb[:, c * _VCH:(c + 1) * _VCH],
                             (((1,), (0,)), ((), ())),
                             preferred_element_type=jnp.float32)
        mns.append(jnp.min(gc, axis=1, keepdims=True))
    mn = jnp.reshape(jnp.transpose(_tree_min(mns)), (1, _TM))

    p2 = jnp.sum(pft * pft, axis=0, keepdims=True)
    validf = 1.0 - jnp.all(pft == 0.0, axis=0,
                           keepdims=True).astype(jnp.float32)
    d = jnp.maximum(mn + p2, 0.0) * validf
    sd = jnp.sum(d)
    cn = jnp.sum(validf)
    lane = lax.broadcasted_iota(jnp.int32, (8, 128), 1)
    part = jnp.where(lane == 0, sd, jnp.where(lane == 1, cn, 0.0))

    t = pl.program_id(1)

    @pl.when(t == 0)
    def _():
        out_ref[0] = part

    @pl.when(t != 0)
    def _():
        out_ref[0] = out_ref[0] + part


_tc_call = pl.pallas_call(
    _tc_body,
    grid=(_B, _PT),
    in_specs=[
        pl.BlockSpec((1, 3, _N), lambda b, t: (b, 0, 0)),
        pl.BlockSpec((1, 3, _TM), lambda b, t: (b, 0, t)),
    ],
    out_specs=pl.BlockSpec((1, 8, 128), lambda b, t: (b, 0, 0)),
    out_shape=jax.ShapeDtypeStruct((_B, 8, 128), jnp.float32),
)


def _sc_body(vert_hbm, pc_hbm, out_hbm,
             vx, vy, vz, v2s, px, py, pz, macc, outv):
    c = lax.axis_index("c")
    s = lax.axis_index("s")
    wid = s * _NC + c
    b = wid // _CHUNKS
    chunk = wid % _CHUNKS

    pltpu.sync_copy(vert_hbm.at[pl.ds((b * 3 + 0) * _N, _N)], vx)
    pltpu.sync_copy(vert_hbm.at[pl.ds((b * 3 + 1) * _N, _N)], vy)
    pltpu.sync_copy(vert_hbm.at[pl.ds((b * 3 + 2) * _N, _N)], vz)
    base = _TC_PTS + chunk * _NPTS
    pltpu.sync_copy(pc_hbm.at[pl.ds((b * 3 + 0) * _M + base, _NPTS)], px)
    pltpu.sync_copy(pc_hbm.at[pl.ds((b * 3 + 1) * _M + base, _NPTS)], py)
    pltpu.sync_copy(pc_hbm.at[pl.ds((b * 3 + 2) * _M + base, _NPTS)], pz)

    def v2_body(i, _):
        o = i * _L
        xv = vx[pl.ds(o, _L)]
        yv = vy[pl.ds(o, _L)]
        zv = vz[pl.ds(o, _L)]
        v2s[pl.ds(o, _L)] = xv * xv + yv * yv + zv * zv
        vx[pl.ds(o, _L)] = _round_bf16(xv)
        vy[pl.ds(o, _L)] = _round_bf16(yv)
        vz[pl.ds(o, _L)] = _round_bf16(zv)
        return 0
    lax.fori_loop(0, _N // _L, v2_body, 0, unroll=4)

    inf16 = jnp.full((_L,), jnp.inf, jnp.float32)

    def init_body(p, _):
        macc[pl.ds(p * _L, _L)] = inf16
        return 0
    lax.fori_loop(0, _NPTS, init_body, 0, unroll=4)

    def blk_body(blk, _):
        vbase = blk * (_VBLK * _L)
        vxs = [vx[pl.ds(vbase + j * _L, _L)] for j in range(_VBLK)]
        vys = [vy[pl.ds(vbase + j * _L, _L)] for j in range(_VBLK)]
        vzs = [vz[pl.ds(vbase + j * _L, _L)] for j in range(_VBLK)]
        v2v = [v2s[pl.ds(vbase + j * _L, _L)] for j in range(_VBLK)]

        def p_body(pg, _):
            o = pg * _L
            axv = -2.0 * _round_bf16(px[pl.ds(o, _L)])
            ayv = -2.0 * _round_bf16(py[pl.ds(o, _L)])
            azv = -2.0 * _round_bf16(pz[pl.ds(o, _L)])
            for u in range(_L):
                ax = axv[u]
                ay = ayv[u]
                az = azv[u]
                m = macc[pl.ds((o + u) * _L, _L)]
                ts = [(v2v[j] + ax * vxs[j]) + (ay * vys[j] + az * vzs[j])
                      for j in range(_VBLK)]
                macc[pl.ds((o + u) * _L, _L)] = jnp.minimum(m, _tree_min(ts))
            return 0
        lax.fori_loop(0, _NPTS // _L, p_body, 0)
        return 0
    lax.fori_loop(0, _NBLK, blk_body, 0)

    def red_body(pg, carry):
        sdv, cnv = carry
        o = pg * _L
        pxv = px[pl.ds(o, _L)]
        pyv = py[pl.ds(o, _L)]
        pzv = pz[pl.ds(o, _L)]
        p2v = pxv * pxv + pyv * pyv + pzv * pzv
        valid = jnp.logical_not((pxv == 0.0) & (pyv == 0.0) & (pzv == 0.0))
        mfv = jnp.where(valid, 1.0, 0.0).astype(jnp.float32)
        lane = lax.iota(jnp.int32, _L)
        mnv = jnp.full((_L,), 0.0, jnp.float32)
        for u in range(_L):
            m = macc[pl.ds((o + u) * _L, _L)]
            mnv = jnp.where(lane == u, jnp.sort(m)[0], mnv)
        dv = jnp.maximum(mnv + p2v, 0.0)
        return sdv + dv * mfv, cnv + mfv
    zero = jnp.zeros((_L,), jnp.float32)
    sdv, cnv = lax.fori_loop(0, _NPTS // _L, red_body, (zero, zero))

    outv[0, :] = sdv
    outv[1, :] = cnv
    pltpu.sync_copy(outv, out_hbm.at[wid])


if _SC_PTS:
    _sc_call = functools.partial(
        pl.kernel,
        out_type=jax.ShapeDtypeStruct((_NW, 2, _L), jnp.float32),
        mesh=plsc.VectorSubcoreMesh(core_axis_name="c", subcore_axis_name="s",
                                    num_cores=_NC, num_subcores=_NS),
        compiler_params=pltpu.CompilerParams(needs_layout_passes=False),
        scratch_types=[
            pltpu.VMEM((_N,), jnp.float32),
            pltpu.VMEM((_N,), jnp.float32),
            pltpu.VMEM((_N,), jnp.float32),
            pltpu.VMEM((_N,), jnp.float32),
            pltpu.VMEM((_NPTS,), jnp.float32),
            pltpu.VMEM((_NPTS,), jnp.float32),
            pltpu.VMEM((_NPTS,), jnp.float32),
            pltpu.VMEM((_NPTS * _L,), jnp.float32),
            pltpu.VMEM((2, _L), jnp.float32),
        ],
    )(_sc_body)


def kernel(vertices, pc):
    tc = _tc_call(vertices, pc)
    sd = tc[:, 0, 0]
    cn = tc[:, 0, 1]

    if _SC_PTS:
        parts = _sc_call(vertices.reshape(-1), pc.reshape(-1))
        parts = parts.reshape(_B, _CHUNKS, 2, _L)
        sd = sd + jnp.sum(parts[:, :, 0, :], axis=(1, 2))
        cn = cn + jnp.sum(parts[:, :, 1, :], axis=(1, 2))

    loss = sd / jnp.maximum(cn, 1.0)
    return jnp.mean(loss)


# speedup vs baseline: 1.0002x; 1.0002x over previous
import functools

import jax
import jax.numpy as jnp
from jax import lax
from jax.experimental import pallas as pl
from jax.experimental.pallas import tpu as pltpu
from jax.experimental.pallas import tpu_sc as plsc

_NC = 2
_NS = 16
_NW = _NC * _NS
_L = 16

_B = 4
_N = 4096
_M = 4096

_TC_PTS = 3584
_SC_PTS = _M - _TC_PTS

_TM = 896
_PT = _TC_PTS // _TM
_VCH = 1024

_CHUNKS = _NW // _B
_NPTS = _SC_PTS // _CHUNKS if _SC_PTS else 0
_VBLK = 8
_NBLK = _N // (_L * _VBLK)


def _round_bf16(v):
    bits = lax.bitcast_convert_type(v, jnp.int32)
    lsb = lax.shift_right_logical(bits, 16) & 1
    rounded = (bits + 0x7FFF + lsb) & jnp.int32(-65536)
    return lax.bitcast_convert_type(rounded, jnp.float32)


def _tree_min(vals):
    while len(vals) > 1:
        vals = [jnp.minimum(vals[2 * i], vals[2 * i + 1])
                for i in range(len(vals) // 2)] + vals[len(vals) & ~1:]
    return vals[0]


def _tc_body(vf_ref, pf_ref, out_ref):
    vff = vf_ref[0]
    pft = pf_ref[0]
    vrr = _round_bf16(vff)
    prr = _round_bf16(pft)

    v2 = jnp.sum(vff * vff, axis=0, keepdims=True)

    v2hi = _round_bf16(v2)
    v2lo = v2 - v2hi
    vb = jnp.concatenate([vrr, v2hi, v2lo],
                         axis=0).astype(jnp.bfloat16)
    pb_rows = jnp.concatenate([-2.0 * prr,
                               jnp.ones((2, _TM), jnp.float32)], axis=0)
    pb = jnp.transpose(pb_rows).astype(jnp.bfloat16)
    mns = []
    for c in range(_N // _VCH):
        gc = lax.dot_general(pb, vb[:, c * _VCH:(c + 1) * _VCH],
                             (((1,), (0,)), ((), ())),
                             preferred_element_type=jnp.float32)
        mns.append(jnp.min(gc, axis=1, keepdims=True))
    mn = jnp.reshape(jnp.transpose(_tree_min(mns)), (1, _TM))

    p2 = jnp.sum(pft * pft, axis=0, keepdims=True)
    validf = 1.0 - jnp.all(pft == 0.0, axis=0,
                           keepdims=True).astype(jnp.float32)
    d = jnp.maximum(mn + p2, 0.0) * validf
    sd = jnp.sum(d)
    cn = jnp.sum(validf)
    lane = lax.broadcasted_iota(jnp.int32, (8, 128), 1)
    part = jnp.where(lane == 0, sd, jnp.where(lane == 1, cn, 0.0))

    t = pl.program_id(1)

    @pl.when(t == 0)
    def _():
        out_ref[0] = part

    @pl.when(t != 0)
    def _():
        out_ref[0] = out_ref[0] + part


_tc_call = pl.pallas_call(
    _tc_body,
    grid=(_B, _PT),
    in_specs=[
        pl.BlockSpec((1, 3, _N), lambda b, t: (b, 0, 0)),
        pl.BlockSpec((1, 3, _TM), lambda b, t: (b, 0, t)),
    ],
    out_specs=pl.BlockSpec((1, 8, 128), lambda b, t: (b, 0, 0)),
    out_shape=jax.ShapeDtypeStruct((_B, 8, 128), jnp.float32),
)


def _sc_body(vert_hbm, pc_hbm, out_hbm,
             vx, vy, vz, v2s, px, py, pz, macc, outv):
    c = lax.axis_index("c")
    s = lax.axis_index("s")
    wid = s * _NC + c
    b = wid // _CHUNKS
    chunk = wid % _CHUNKS

    pltpu.sync_copy(vert_hbm.at[pl.ds((b * 3 + 0) * _N, _N)], vx)
    pltpu.sync_copy(vert_hbm.at[pl.ds((b * 3 + 1) * _N, _N)], vy)
    pltpu.sync_copy(vert_hbm.at[pl.ds((b * 3 + 2) * _N, _N)], vz)
    base = _TC_PTS + chunk * _NPTS
    pltpu.sync_copy(pc_hbm.at[pl.ds((b * 3 + 0) * _M + base, _NPTS)], px)
    pltpu.sync_copy(pc_hbm.at[pl.ds((b * 3 + 1) * _M + base, _NPTS)], py)
    pltpu.sync_copy(pc_hbm.at[pl.ds((b * 3 + 2) * _M + base, _NPTS)], pz)

    def v2_body(i, _):
        o = i * _L
        xv = vx[pl.ds(o, _L)]
        yv = vy[pl.ds(o, _L)]
        zv = vz[pl.ds(o, _L)]
        v2s[pl.ds(o, _L)] = xv * xv + yv * yv + zv * zv
        vx[pl.ds(o, _L)] = _round_bf16(xv)
        vy[pl.ds(o, _L)] = _round_bf16(yv)
        vz[pl.ds(o, _L)] = _round_bf16(zv)
        return 0
    lax.fori_loop(0, _N // _L, v2_body, 0, unroll=4)

    inf16 = jnp.full((_L,), jnp.inf, jnp.float32)

    def init_body(p, _):
        macc[pl.ds(p * _L, _L)] = inf16
        return 0
    lax.fori_loop(0, _NPTS, init_body, 0, unroll=4)

    def blk_body(blk, _):
        vbase = blk * (_VBLK * _L)
        vxs = [vx[pl.ds(vbase + j * _L, _L)] for j in range(_VBLK)]
        vys = [vy[pl.ds(vbase + j * _L, _L)] for j in range(_VBLK)]
        vzs = [vz[pl.ds(vbase + j * _L, _L)] for j in range(_VBLK)]
        v2v = [v2s[pl.ds(vbase + j * _L, _L)] for j in range(_VBLK)]

        def p_body(pg, _):
            o = pg * _L
            axv = -2.0 * _round_bf16(px[pl.ds(o, _L)])
            ayv = -2.0 * _round_bf16(py[pl.ds(o, _L)])
            azv = -2.0 * _round_bf16(pz[pl.ds(o, _L)])
            for u in range(_L):
                ax = axv[u]
                ay = ayv[u]
                az = azv[u]
                m = macc[pl.ds((o + u) * _L, _L)]
                ts = [(v2v[j] + ax * vxs[j]) + (ay * vys[j] + az * vzs[j])
                      for j in range(_VBLK)]
                macc[pl.ds((o + u) * _L, _L)] = jnp.minimum(m, _tree_min(ts))
            return 0
        lax.fori_loop(0, _NPTS // _L, p_body, 0)
        return 0
    lax.fori_loop(0, _NBLK, blk_body, 0)

    def red_body(pg, carry):
        sdv, cnv = carry
        o = pg * _L
        pxv = px[pl.ds(o, _L)]
        pyv = py[pl.ds(o, _L)]
        pzv = pz[pl.ds(o, _L)]
        p2v = pxv * pxv + pyv * pyv + pzv * pzv
        valid = jnp.logical_not((pxv == 0.0) & (pyv == 0.0) & (pzv == 0.0))
        mfv = jnp.where(valid, 1.0, 0.0).astype(jnp.float32)
        lane = lax.iota(jnp.int32, _L)
        mnv = jnp.full((_L,), 0.0, jnp.float32)
        for u in range(_L):
            m = macc[pl.ds((o + u) * _L, _L)]
            mnv = jnp.where(lane == u, jnp.sort(m)[0], mnv)
        dv = jnp.maximum(mnv + p2v, 0.0)
        return sdv + dv * mfv, cnv + mfv
    zero = jnp.zeros((_L,), jnp.float32)
    sdv, cnv = lax.fori_loop(0, _NPTS // _L, red_body, (zero, zero))

    outv[0, :] = sdv
    outv[1, :] = cnv
    pltpu.sync_copy(outv, out_hbm.at[wid])


if _SC_PTS:
    _sc_call = functools.partial(
        pl.kernel,
        out_type=jax.ShapeDtypeStruct((_NW, 2, _L), jnp.float32),
        mesh=plsc.VectorSubcoreMesh(core_axis_name="c", subcore_axis_name="s",
                                    num_cores=_NC, num_subcores=_NS),
        compiler_params=pltpu.CompilerParams(needs_layout_passes=False),
        scratch_types=[
            pltpu.VMEM((_N,), jnp.float32),
            pltpu.VMEM((_N,), jnp.float32),
            pltpu.VMEM((_N,), jnp.float32),
            pltpu.VMEM((_N,), jnp.float32),
            pltpu.VMEM((_NPTS,), jnp.float32),
            pltpu.VMEM((_NPTS,), jnp.float32),
            pltpu.VMEM((_NPTS,), jnp.float32),
            pltpu.VMEM((_NPTS * _L,), jnp.float32),
            pltpu.VMEM((2, _L), jnp.float32),
        ],
    )(_sc_body)


def kernel(vertices, pc):
    if _SC_PTS:
        parts = _sc_call(vertices.reshape(-1), pc.reshape(-1))

    tc = _tc_call(vertices, pc)
    sd = tc[:, 0, 0]
    cn = tc[:, 0, 1]

    if _SC_PTS:
        parts = parts.reshape(_B, _CHUNKS, 2, _L)
        sd = sd + jnp.sum(parts[:, :, 0, :], axis=(1, 2))
        cn = cn + jnp.sum(parts[:, :, 1, :], axis=(1, 2))

    loss = sd / jnp.maximum(cn, 1.0)
    return jnp.mean(loss)


# speedup vs baseline: 1.3723x; 1.3720x over previous
import functools

import jax
import jax.numpy as jnp
from jax import lax
from jax.experimental import pallas as pl
from jax.experimental.pallas import tpu as pltpu
from jax.experimental.pallas import tpu_sc as plsc

_NC = 2
_NS = 16
_NW = _NC * _NS
_L = 16

_B = 4
_N = 4096
_M = 4096

_TC_PTS = 4096
_SC_PTS = _M - _TC_PTS

_TM = 4096
_PT = _TC_PTS // _TM
_VCH = 2048

_CHUNKS = _NW // _B
_NPTS = _SC_PTS // _CHUNKS if _SC_PTS else 0
_VBLK = 8
_NBLK = _N // (_L * _VBLK)


def _round_bf16(v):
    bits = lax.bitcast_convert_type(v, jnp.int32)
    lsb = lax.shift_right_logical(bits, 16) & 1
    rounded = (bits + 0x7FFF + lsb) & jnp.int32(-65536)
    return lax.bitcast_convert_type(rounded, jnp.float32)


def _tree_min(vals):
    while len(vals) > 1:
        vals = [jnp.minimum(vals[2 * i], vals[2 * i + 1])
                for i in range(len(vals) // 2)] + vals[len(vals) & ~1:]
    return vals[0]


def _tc_body(vf_ref, pf_ref, out_ref):
    vff = vf_ref[0]
    pft = pf_ref[0]
    vrr = _round_bf16(vff)
    prr = _round_bf16(pft)

    v2 = jnp.sum(vff * vff, axis=0, keepdims=True)

    v2hi = _round_bf16(v2)
    v2lo = v2 - v2hi
    vb = jnp.concatenate([vrr, v2hi, v2lo],
                         axis=0).astype(jnp.bfloat16)
    pb_rows = jnp.concatenate([-2.0 * prr,
                               jnp.ones((2, _TM), jnp.float32)], axis=0)
    pb = jnp.transpose(pb_rows).astype(jnp.bfloat16)
    mns = []
    for c in range(_N // _VCH):
        gc = lax.dot_general(pb, vb[:, c * _VCH:(c + 1) * _VCH],
                             (((1,), (0,)), ((), ())),
                             preferred_element_type=jnp.float32)
        mns.append(jnp.min(gc, axis=1, keepdims=True))
    mn = jnp.reshape(jnp.transpose(_tree_min(mns)), (1, _TM))

    p2 = jnp.sum(pft * pft, axis=0, keepdims=True)
    validf = 1.0 - jnp.all(pft == 0.0, axis=0,
                           keepdims=True).astype(jnp.float32)
    d = jnp.maximum(mn + p2, 0.0) * validf
    sd = jnp.sum(d)
    cn = jnp.sum(validf)
    lane = lax.broadcasted_iota(jnp.int32, (8, 128), 1)
    part = jnp.where(lane == 0, sd, jnp.where(lane == 1, cn, 0.0))

    t = pl.program_id(1)

    @pl.when(t == 0)
    def _():
        out_ref[0] = part

    @pl.when(t != 0)
    def _():
        out_ref[0] = out_ref[0] + part


_tc_call = pl.pallas_call(
    _tc_body,
    grid=(_B, _PT),
    in_specs=[
        pl.BlockSpec((1, 3, _N), lambda b, t: (b, 0, 0)),
        pl.BlockSpec((1, 3, _TM), lambda b, t: (b, 0, t)),
    ],
    out_specs=pl.BlockSpec((1, 8, 128), lambda b, t: (b, 0, 0)),
    out_shape=jax.ShapeDtypeStruct((_B, 8, 128), jnp.float32),
)


def _sc_body(vert_hbm, pc_hbm, out_hbm,
             vx, vy, vz, v2s, px, py, pz, macc, outv):
    c = lax.axis_index("c")
    s = lax.axis_index("s")
    wid = s * _NC + c
    b = wid // _CHUNKS
    chunk = wid % _CHUNKS

    pltpu.sync_copy(vert_hbm.at[pl.ds((b * 3 + 0) * _N, _N)], vx)
    pltpu.sync_copy(vert_hbm.at[pl.ds((b * 3 + 1) * _N, _N)], vy)
    pltpu.sync_copy(vert_hbm.at[pl.ds((b * 3 + 2) * _N, _N)], vz)
    base = _TC_PTS + chunk * _NPTS
    pltpu.sync_copy(pc_hbm.at[pl.ds((b * 3 + 0) * _M + base, _NPTS)], px)
    pltpu.sync_copy(pc_hbm.at[pl.ds((b * 3 + 1) * _M + base, _NPTS)], py)
    pltpu.sync_copy(pc_hbm.at[pl.ds((b * 3 + 2) * _M + base, _NPTS)], pz)

    def v2_body(i, _):
        o = i * _L
        xv = vx[pl.ds(o, _L)]
        yv = vy[pl.ds(o, _L)]
        zv = vz[pl.ds(o, _L)]
        v2s[pl.ds(o, _L)] = xv * xv + yv * yv + zv * zv
        vx[pl.ds(o, _L)] = _round_bf16(xv)
        vy[pl.ds(o, _L)] = _round_bf16(yv)
        vz[pl.ds(o, _L)] = _round_bf16(zv)
        return 0
    lax.fori_loop(0, _N // _L, v2_body, 0, unroll=4)

    inf16 = jnp.full((_L,), jnp.inf, jnp.float32)

    def init_body(p, _):
        macc[pl.ds(p * _L, _L)] = inf16
        return 0
    lax.fori_loop(0, _NPTS, init_body, 0, unroll=4)

    def blk_body(blk, _):
        vbase = blk * (_VBLK * _L)
        vxs = [vx[pl.ds(vbase + j * _L, _L)] for j in range(_VBLK)]
        vys = [vy[pl.ds(vbase + j * _L, _L)] for j in range(_VBLK)]
        vzs = [vz[pl.ds(vbase + j * _L, _L)] for j in range(_VBLK)]
        v2v = [v2s[pl.ds(vbase + j * _L, _L)] for j in range(_VBLK)]

        def p_body(pg, _):
            o = pg * _L
            axv = -2.0 * _round_bf16(px[pl.ds(o, _L)])
            ayv = -2.0 * _round_bf16(py[pl.ds(o, _L)])
            azv = -2.0 * _round_bf16(pz[pl.ds(o, _L)])
            for u in range(_L):
                ax = axv[u]
                ay = ayv[u]
                az = azv[u]
                m = macc[pl.ds((o + u) * _L, _L)]
                ts = [(v2v[j] + ax * vxs[j]) + (ay * vys[j] + az * vzs[j])
                      for j in range(_VBLK)]
                macc[pl.ds((o + u) * _L, _L)] = jnp.minimum(m, _tree_min(ts))
            return 0
        lax.fori_loop(0, _NPTS // _L, p_body, 0)
        return 0
    lax.fori_loop(0, _NBLK, blk_body, 0)

    def red_body(pg, carry):
        sdv, cnv = carry
        o = pg * _L
        pxv = px[pl.ds(o, _L)]
        pyv = py[pl.ds(o, _L)]
        pzv = pz[pl.ds(o, _L)]
        p2v = pxv * pxv + pyv * pyv + pzv * pzv
        valid = jnp.logical_not((pxv == 0.0) & (pyv == 0.0) & (pzv == 0.0))
        mfv = jnp.where(valid, 1.0, 0.0).astype(jnp.float32)
        lane = lax.iota(jnp.int32, _L)
        mnv = jnp.full((_L,), 0.0, jnp.float32)
        for u in range(_L):
            m = macc[pl.ds((o + u) * _L, _L)]
            mnv = jnp.where(lane == u, jnp.sort(m)[0], mnv)
        dv = jnp.maximum(mnv + p2v, 0.0)
        return sdv + dv * mfv, cnv + mfv
    zero = jnp.zeros((_L,), jnp.float32)
    sdv, cnv = lax.fori_loop(0, _NPTS // _L, red_body, (zero, zero))

    outv[0, :] = sdv
    outv[1, :] = cnv
    pltpu.sync_copy(outv, out_hbm.at[wid])


if _SC_PTS:
    _sc_call = functools.partial(
        pl.kernel,
        out_type=jax.ShapeDtypeStruct((_NW, 2, _L), jnp.float32),
        mesh=plsc.VectorSubcoreMesh(core_axis_name="c", subcore_axis_name="s",
                                    num_cores=_NC, num_subcores=_NS),
        compiler_params=pltpu.CompilerParams(needs_layout_passes=False),
        scratch_types=[
            pltpu.VMEM((_N,), jnp.float32),
            pltpu.VMEM((_N,), jnp.float32),
            pltpu.VMEM((_N,), jnp.float32),
            pltpu.VMEM((_N,), jnp.float32),
            pltpu.VMEM((_NPTS,), jnp.float32),
            pltpu.VMEM((_NPTS,), jnp.float32),
            pltpu.VMEM((_NPTS,), jnp.float32),
            pltpu.VMEM((_NPTS * _L,), jnp.float32),
            pltpu.VMEM((2, _L), jnp.float32),
        ],
    )(_sc_body)


def kernel(vertices, pc):
    if _SC_PTS:
        parts = _sc_call(vertices.reshape(-1), pc.reshape(-1))

    tc = _tc_call(vertices, pc)
    sd = tc[:, 0, 0]
    cn = tc[:, 0, 1]

    if _SC_PTS:
        parts = parts.reshape(_B, _CHUNKS, 2, _L)
        sd = sd + jnp.sum(parts[:, :, 0, :], axis=(1, 2))
        cn = cn + jnp.sum(parts[:, :, 1, :], axis=(1, 2))

    loss = sd / jnp.maximum(cn, 1.0)
    return jnp.mean(loss)
